# baseline (speedup 1.0000x reference)
import functools

import jax
import jax.numpy as jnp
from jax import lax
from jax.experimental import pallas as pl
from jax.experimental.pallas import tpu as pltpu
from jax.experimental.pallas import tpu_sc as plsc

BATCH, HIST = 4096, 20
VOCAB, D_IN = 1000, 64

_NC, _NS = 2, 16
_NW = _NC * _NS
_B_TOT = BATCH * HIST
_ROWS_PER_W = _B_TOT // _NW
_VPAD = 1024
_CHUNK = 40
_NCHUNKS = _ROWS_PER_W // _CHUNK


def _table_body(emb_ref, w_ref, b_ref, out_ref):
    out_ref[...] = lax.dot_general(
        emb_ref[...], w_ref[...],
        dimension_numbers=(((1,), (1,)), ((), ())),
        preferred_element_type=jnp.float32,
    ) + b_ref[...]


def _sc_gather_body(table_hbm, idx_hbm, out_hbm, idx_v, buf, gsem):
    wid = lax.axis_index("s") * _NC + lax.axis_index("c")
    base = wid * _ROWS_PER_W
    pltpu.sync_copy(idx_hbm.at[pl.ds(base, _ROWS_PER_W)], idx_v)

    def step(c, carry):
        off = c * _CHUNK
        pltpu.async_copy(
            table_hbm.at[idx_v.at[pl.ds(off, _CHUNK)]], buf, gsem
        ).wait()
        pltpu.sync_copy(buf, out_hbm.at[pl.ds(base + off, _CHUNK)])
        return carry

    lax.fori_loop(0, _NCHUNKS, step, 0)


_sc_gather = functools.partial(
    pl.kernel,
    out_type=jax.ShapeDtypeStruct((_B_TOT, VOCAB), jnp.float32),
    mesh=plsc.VectorSubcoreMesh(
        core_axis_name="c", subcore_axis_name="s",
        num_cores=_NC, num_subcores=_NS),
    scratch_types=[
        pltpu.VMEM((_ROWS_PER_W,), jnp.int32),
        pltpu.VMEM((_CHUNK, VOCAB), jnp.float32),
        pltpu.SemaphoreType.DMA,
    ],
    compiler_params=pltpu.CompilerParams(use_tc_tiling_on_sc=False),
)(_sc_gather_body)


def kernel(x, emb_table, W_head, b_head):
    table = pl.pallas_call(
        _table_body,
        out_shape=jax.ShapeDtypeStruct((VOCAB, VOCAB), jnp.float32),
    )(emb_table, W_head, b_head.reshape(1, VOCAB))
    out = _sc_gather(table, x.reshape(-1))
    return out.reshape(BATCH, HIST, VOCAB)


# speedup vs baseline: 1.1969x; 1.1969x over previous
import functools

import jax
import jax.numpy as jnp
from jax import lax
from jax.experimental import pallas as pl
from jax.experimental.pallas import tpu as pltpu
from jax.experimental.pallas import tpu_sc as plsc

BATCH, HIST = 4096, 20
VOCAB, D_IN = 1000, 64

_NC, _NS = 2, 16
_NW = _NC * _NS
_B_TOT = BATCH * HIST
_ROWS_PER_W = _B_TOT // _NW
_VPAD = 1024
_CHUNK = 32
_NCHUNKS = _ROWS_PER_W // _CHUNK


def _table_body(emb_ref, w_ref, b_ref, out_ref):
    out_ref[...] = lax.dot_general(
        emb_ref[...], w_ref[...],
        dimension_numbers=(((1,), (1,)), ((), ())),
        preferred_element_type=jnp.float32,
    ) + b_ref[...]


_LOAD_ROWS = 63


def _sc_gather_body(table_hbm, idx_hbm, out_hbm,
                    table_sh, idx_v, buf0, buf1, gsem0, gsem1, wsem0, wsem1):
    sid = lax.axis_index("s")
    wid = sid * _NC + lax.axis_index("c")
    base = wid * _ROWS_PER_W

    r0 = jnp.minimum(sid * _LOAD_ROWS, VOCAB - _LOAD_ROWS)
    pltpu.sync_copy(table_hbm.at[pl.ds(r0, _LOAD_ROWS)],
                    table_sh.at[pl.ds(r0, _LOAD_ROWS)])
    pltpu.sync_copy(idx_hbm.at[pl.ds(base, _ROWS_PER_W)], idx_v)
    plsc.subcore_barrier()

    def gather(c, buf, sem):
        return pltpu.make_async_copy(
            table_sh.at[idx_v.at[pl.ds(c * _CHUNK, _CHUNK)]], buf, sem)

    def write(c, buf, sem):
        return pltpu.make_async_copy(
            buf, out_hbm.at[pl.ds(base + c * _CHUNK, _CHUNK)], sem)

    gather(0, buf0, gsem0).start()

    def step(g, carry):
        c0 = 2 * g
        gather(0, buf0, gsem0).wait()

        @pl.when(g > 0)
        def _():
            write(0, buf1, wsem1).wait()

        gather(c0 + 1, buf1, gsem1).start()
        write(c0, buf0, wsem0).start()
        gather(0, buf1, gsem1).wait()
        write(0, buf0, wsem0).wait()

        @pl.when(g + 1 < _NCHUNKS // 2)
        def _():
            gather(c0 + 2, buf0, gsem0).start()

        write(c0 + 1, buf1, wsem1).start()
        return carry

    lax.fori_loop(0, _NCHUNKS // 2, step, 0)
    write(0, buf1, wsem1).wait()


_sc_gather = functools.partial(
    pl.kernel,
    out_type=jax.ShapeDtypeStruct((_B_TOT, VOCAB), jnp.float32),
    mesh=plsc.VectorSubcoreMesh(
        core_axis_name="c", subcore_axis_name="s",
        num_cores=_NC, num_subcores=_NS),
    scratch_types=[
        pltpu.VMEM_SHARED((VOCAB, VOCAB), jnp.float32),
        pltpu.VMEM((_ROWS_PER_W,), jnp.int32),
        pltpu.VMEM((_CHUNK, VOCAB), jnp.float32),
        pltpu.VMEM((_CHUNK, VOCAB), jnp.float32),
        pltpu.SemaphoreType.DMA,
        pltpu.SemaphoreType.DMA,
        pltpu.SemaphoreType.DMA,
        pltpu.SemaphoreType.DMA,
    ],
    compiler_params=pltpu.CompilerParams(use_tc_tiling_on_sc=False),
)(_sc_gather_body)


def kernel(x, emb_table, W_head, b_head):
    table = pl.pallas_call(
        _table_body,
        out_shape=jax.ShapeDtypeStruct((VOCAB, VOCAB), jnp.float32),
    )(emb_table, W_head, b_head.reshape(1, VOCAB))
    out = _sc_gather(table, x.reshape(-1))
    return out.reshape(BATCH, HIST, VOCAB)
